# baseline (speedup 1.0000x reference)
import functools

import jax
import jax.numpy as jnp
from jax import lax
from jax.experimental import pallas as pl
from jax.experimental.pallas import tpu as pltpu
from jax.experimental.pallas import tpu_sc as plsc

_BATCH = 4096
_MAX_LEN = 200
_EMBED = 32
_B = _BATCH * _MAX_LEN
_NC = 2
_NS = 16
_NW = _NC * _NS
_BPW = _B // _NW
_CHUNK = 1600
_NCHUNK = _BPW // _CHUNK


@jax.jit
def _embedding_sc(idx_flat, table):
    mesh = plsc.VectorSubcoreMesh(core_axis_name="c", subcore_axis_name="s")

    nstream = 8
    sub = 200
    block = nstream * sub
    nround = _BPW // block

    @functools.partial(
        pl.kernel,
        mesh=mesh,
        out_type=jax.ShapeDtypeStruct((_B, _EMBED), jnp.float32),
        scratch_types=[
            pltpu.VMEM((_BPW,), jnp.int32),
            pltpu.VMEM((2, block, _EMBED), jnp.float32),
            pltpu.SemaphoreType.DMA((2,)),
            pltpu.SemaphoreType.DMA((2,)),
        ],
        compiler_params=pltpu.CompilerParams(use_tc_tiling_on_sc=False),
    )
    def k(idx_hbm, table_hbm, out_hbm, idx_v, rows_v, gsem, wsem):
        wid = lax.axis_index("s") * _NC + lax.axis_index("c")
        base = wid * _BPW
        pltpu.sync_copy(idx_hbm.at[pl.ds(base, _BPW)], idx_v)

        def g_desc(r, j, b):
            return pltpu.make_async_copy(
                table_hbm.at[idx_v.at[pl.ds(r * block + j * sub, sub)]],
                rows_v.at[b].at[pl.ds(j * sub, sub)], gsem.at[b])

        def w_desc(r, b):
            return pltpu.make_async_copy(
                rows_v.at[b], out_hbm.at[pl.ds(base + r * block, block)],
                wsem.at[b])

        for r in range(nround):
            b = r % 2
            if r >= 2:
                w_desc(r - 2, b).wait()
            for j in range(nstream):
                g_desc(r, j, b).start()
            for j in range(nstream):
                g_desc(r, j, b).wait()
            w_desc(r, b).start()
        w_desc(nround - 2, (nround - 2) % 2).wait()
        w_desc(nround - 1, (nround - 1) % 2).wait()

    return k(idx_flat, table)


def kernel(inputs, table):
    idx_flat = inputs.reshape(-1).astype(jnp.int32)
    out = _embedding_sc(idx_flat, table)
    return out.reshape(_BATCH, _MAX_LEN, _EMBED)
